# baseline (speedup 1.0000x reference)
import functools

import jax
import jax.numpy as jnp
import numpy as np
from jax import lax
from jax.experimental import pallas as pl
from jax.experimental.pallas import tpu as pltpu
from jax.experimental.pallas import tpu_sc as plsc

_NUM_CLASSES = 1_000_000
_EMB = 64
_NUM_SAMPLE = 10_000
_B = 1024
_S = 64.0
_M = 0.5
_COS_M = float(np.cos(_M))
_SIN_M = float(np.sin(_M))

_NPAD = 10_240
_NROWS = _NPAD + _B
_BLK = 512
_NBLK = _NPAD // _BLK
_NCAND = _NUM_SAMPLE + _B

_NW = 32
_RPW = _NROWS // _NW
_CHUNK = 88
_NCHUNK = _RPW // _CHUNK
_PAIRW = 2 * _EMB


@functools.cache
def _build_candidates() -> np.ndarray:
    with jax.ensure_compile_time_eval():
        perm = np.asarray(
            jax.random.uniform(
                jax.random.key(42), (_NUM_CLASSES,), dtype=jnp.float32
            )
        )
    return np.argsort(-perm, kind="stable")[:_NCAND].astype(np.int32)


def _sc_gather_body(table_hbm, idx_hbm, out_hbm, idx_v, rows_v, sem):
    wid = lax.axis_index("s") * 2 + lax.axis_index("c")
    pltpu.sync_copy(idx_hbm.at[pl.ds(wid * _NCHUNK, _NCHUNK)], idx_v)
    copies = []
    for c in range(_NCHUNK):
        copies.append(
            pltpu.async_copy(
                table_hbm.at[idx_v.at[c]],
                rows_v.at[pl.ds(c * _CHUNK, _CHUNK)],
                sem,
            )
        )
    for cp in copies:
        cp.wait()
    pltpu.sync_copy(rows_v, out_hbm.at[pl.ds(wid * _RPW, _RPW)])


@functools.cache
def _sc_gather():
    return pl.kernel(
        _sc_gather_body,
        out_type=jax.ShapeDtypeStruct((_NROWS, _PAIRW), jnp.float32),
        mesh=plsc.VectorSubcoreMesh(
            core_axis_name="c", subcore_axis_name="s", num_cores=2, num_subcores=16
        ),
        scratch_types=[
            pltpu.VMEM((_NCHUNK, _CHUNK), jnp.int32),
            pltpu.VMEM((_RPW, _PAIRW), jnp.float32),
            pltpu.SemaphoreType.DMA,
        ],
    )


def _tc_body(emb_ref, wneg_ref, pneg_ref, wt_ref, pt_ref, out_ref, ne_ref, m_ref, s_ref):
    pid = pl.program_id(0)

    @pl.when(pid == 0)
    def _init():
        e = emb_ref[...]
        nrm = jnp.sqrt(jnp.sum(e * e, axis=1, keepdims=True))
        ne_ref[...] = e / jnp.maximum(nrm, 1e-12)
        m_ref[...] = jnp.full((_B, 1), -1e30, jnp.float32)
        s_ref[...] = jnp.zeros((_B, 1), jnp.float32)

    wp = wneg_ref[...]
    w = jnp.where(pneg_ref[...] > 0.5, wp[:, _EMB:], wp[:, :_EMB])
    nw = w / jnp.maximum(jnp.sqrt(jnp.sum(w * w, axis=1, keepdims=True)), 1e-12)
    ne = ne_ref[...]
    p = lax.dot_general(
        ne, nw, (((1,), (1,)), ((), ())), preferred_element_type=jnp.float32
    )
    p = jnp.clip(p, -1.0, 1.0) * _S
    col = pid * _BLK + lax.broadcasted_iota(jnp.int32, (_B, _BLK), 1)
    p = jnp.where(col < _NUM_SAMPLE, p, -1e9)
    bm = jnp.max(p, axis=1, keepdims=True)
    m_old = m_ref[...]
    m_new = jnp.maximum(m_old, bm)
    s_ref[...] = s_ref[...] * jnp.exp(m_old - m_new) + jnp.sum(
        jnp.exp(p - m_new), axis=1, keepdims=True
    )
    m_ref[...] = m_new

    @pl.when(pid == _NBLK - 1)
    def _finish():
        wtp = wt_ref[...]
        wt = jnp.where(pt_ref[...] > 0.5, wtp[:, _EMB:], wtp[:, :_EMB])
        nwt = wt / jnp.maximum(
            jnp.sqrt(jnp.sum(wt * wt, axis=1, keepdims=True)), 1e-12
        )
        t = jnp.sum(ne_ref[...] * nwt, axis=1, keepdims=True)
        t = jnp.clip(t, -1.0, 1.0)
        lt = t * _S
        tcl = jnp.clip(t, -1.0 + 1e-7, 1.0 - 1e-7)
        lm = _S * (tcl * _COS_M - jnp.sqrt(1.0 - tcl * tcl) * _SIN_M)
        m = m_ref[...]
        z = s_ref[...] - jnp.exp(lt - m) + jnp.exp(lm - m)
        z = jnp.maximum(z, 1e-30)
        cap = float(-np.log(1e-30))
        loss_rows = jnp.minimum(jnp.log(z) + m - lm, cap)
        out_ref[...] = jnp.sum(loss_rows, axis=0, keepdims=True) / float(_B)


_tc_loss = pl.pallas_call(
    _tc_body,
    grid=(_NBLK,),
    in_specs=[
        pl.BlockSpec((_B, _EMB), lambda i: (0, 0)),
        pl.BlockSpec((_BLK, _PAIRW), lambda i: (i, 0)),
        pl.BlockSpec((_BLK, 1), lambda i: (i, 0)),
        pl.BlockSpec((_B, _PAIRW), lambda i: (0, 0)),
        pl.BlockSpec((_B, 1), lambda i: (0, 0)),
    ],
    out_specs=pl.BlockSpec((1, 1), lambda i: (0, 0)),
    out_shape=jax.ShapeDtypeStruct((1, 1), jnp.float32),
    scratch_shapes=[
        pltpu.VMEM((_B, _EMB), jnp.float32),
        pltpu.VMEM((_B, 1), jnp.float32),
        pltpu.VMEM((_B, 1), jnp.float32),
    ],
)


def kernel(local_embeddings, local_labels, weight):
    labels = local_labels.astype(jnp.int32)
    cand = jnp.asarray(_build_candidates())

    arange_b = jnp.arange(_B, dtype=jnp.int32)
    occ = jnp.zeros((_NUM_CLASSES,), jnp.int32).at[labels].set(1)
    s = jnp.sum(occ)
    padded = s < _B
    zero_was = occ[0]
    forced0 = padded & (zero_was == 0)
    u = s + forced0.astype(jnp.int32)
    is_pos = (occ[cand] == 1) | (forced0 & (cand == 0))
    nneg = _NUM_SAMPLE - u
    rank = jnp.cumsum((~is_pos).astype(jnp.int32)) - 1
    negpos = jnp.where((~is_pos) & (rank < nneg), rank, _NPAD)
    buf = jnp.zeros((_NPAD + 1,), jnp.int32).at[negpos].set(cand, mode="drop")
    slot = jnp.full((_NUM_CLASSES,), 2**30, jnp.int32).at[labels].min(arange_b)
    firstocc = slot[labels] == arange_b
    posrank = jnp.cumsum(firstocc.astype(jnp.int32)) - 1
    pospos = jnp.where(
        firstocc, nneg + forced0.astype(jnp.int32) + posrank, _NPAD
    )
    buf = buf.at[pospos].set(labels, mode="drop")

    gidx = jnp.concatenate([buf[:_NPAD], labels])
    pidx = (gidx >> 1).reshape(_NW * _NCHUNK, _CHUNK)
    par = (gidx & 1).astype(jnp.float32).reshape(_NROWS, 1)
    rows = _sc_gather()(weight.reshape(_NUM_CLASSES // 2, _PAIRW), pidx)

    loss = _tc_loss(
        local_embeddings, rows[:_NPAD], par[:_NPAD], rows[_NPAD:], par[_NPAD:]
    )
    return loss.reshape(())


# speedup vs baseline: 1.3871x; 1.3871x over previous
import functools

import jax
import jax.numpy as jnp
import numpy as np
from jax import lax
from jax.experimental import pallas as pl
from jax.experimental.pallas import tpu as pltpu
from jax.experimental.pallas import tpu_sc as plsc

_NUM_CLASSES = 1_000_000
_EMB = 64
_NUM_SAMPLE = 10_000
_B = 1024
_S = 64.0
_M = 0.5
_COS_M = float(np.cos(_M))
_SIN_M = float(np.sin(_M))

_NPAD = 10_240
_NROWS = _NPAD + _B
_BLK = 512
_NBLK = _NPAD // _BLK
_NCAND = _NUM_SAMPLE + _B

_NW = 32
_RPW = _NROWS // _NW
_CHUNK = 88
_NCHUNK = _RPW // _CHUNK
_PAIRW = 2 * _EMB


@functools.cache
def _build_candidates() -> np.ndarray:
    with jax.ensure_compile_time_eval():
        perm = np.asarray(
            jax.random.uniform(
                jax.random.key(42), (_NUM_CLASSES,), dtype=jnp.float32
            )
        )
    return np.argsort(-perm, kind="stable")[:_NCAND].astype(np.int32)


def _sc_gather_body(table_hbm, idx_hbm, out_hbm, idx_v, rows_v, sem):
    wid = lax.axis_index("s") * 2 + lax.axis_index("c")
    pltpu.sync_copy(idx_hbm.at[wid], idx_v)

    def _group(g, _):
        vec = idx_v[pl.ds(g * 16, 16)]
        copies = []
        for b in range(16):
            r = g * 16 + b
            copies.append(
                pltpu.async_copy(
                    table_hbm.at[pl.ds(vec[b], 1)],
                    rows_v.at[pl.ds(r, 1)],
                    sem,
                )
            )
        for cp in copies:
            cp.wait()
        return _

    lax.fori_loop(0, _RPW // 16, _group, None)
    pltpu.sync_copy(rows_v, out_hbm.at[pl.ds(wid * _RPW, _RPW)])


@functools.cache
def _sc_gather():
    return pl.kernel(
        _sc_gather_body,
        out_type=jax.ShapeDtypeStruct((_NROWS, _EMB), jnp.float32),
        mesh=plsc.VectorSubcoreMesh(
            core_axis_name="c", subcore_axis_name="s", num_cores=2, num_subcores=16
        ),
        scratch_types=[
            pltpu.VMEM((_RPW,), jnp.int32),
            pltpu.VMEM((_RPW, _EMB), jnp.float32),
            pltpu.SemaphoreType.DMA,
        ],
    )


def _tc_body(emb_ref, wneg_ref, wt_ref, out_ref, ne_ref, m_ref, s_ref):
    pid = pl.program_id(0)

    @pl.when(pid == 0)
    def _init():
        e = emb_ref[...]
        nrm = jnp.sqrt(jnp.sum(e * e, axis=1, keepdims=True))
        ne_ref[...] = e / jnp.maximum(nrm, 1e-12)
        m_ref[...] = jnp.full((_B, 1), -1e30, jnp.float32)
        s_ref[...] = jnp.zeros((_B, 1), jnp.float32)

    w = wneg_ref[...]
    nw = w / jnp.maximum(jnp.sqrt(jnp.sum(w * w, axis=1, keepdims=True)), 1e-12)
    ne = ne_ref[...]
    p = lax.dot_general(
        ne, nw, (((1,), (1,)), ((), ())), preferred_element_type=jnp.float32
    )
    p = jnp.clip(p, -1.0, 1.0) * _S
    col = pid * _BLK + lax.broadcasted_iota(jnp.int32, (_B, _BLK), 1)
    p = jnp.where(col < _NUM_SAMPLE, p, -1e9)
    bm = jnp.max(p, axis=1, keepdims=True)
    m_old = m_ref[...]
    m_new = jnp.maximum(m_old, bm)
    s_ref[...] = s_ref[...] * jnp.exp(m_old - m_new) + jnp.sum(
        jnp.exp(p - m_new), axis=1, keepdims=True
    )
    m_ref[...] = m_new

    @pl.when(pid == _NBLK - 1)
    def _finish():
        wt = wt_ref[...]
        nwt = wt / jnp.maximum(
            jnp.sqrt(jnp.sum(wt * wt, axis=1, keepdims=True)), 1e-12
        )
        t = jnp.sum(ne_ref[...] * nwt, axis=1, keepdims=True)
        t = jnp.clip(t, -1.0, 1.0)
        lt = t * _S
        tcl = jnp.clip(t, -1.0 + 1e-7, 1.0 - 1e-7)
        lm = _S * (tcl * _COS_M - jnp.sqrt(1.0 - tcl * tcl) * _SIN_M)
        m = m_ref[...]
        z = s_ref[...] - jnp.exp(lt - m) + jnp.exp(lm - m)
        z = jnp.maximum(z, 1e-30)
        cap = float(-np.log(1e-30))
        loss_rows = jnp.minimum(jnp.log(z) + m - lm, cap)
        out_ref[...] = jnp.sum(loss_rows, axis=0, keepdims=True) / float(_B)


_tc_loss = pl.pallas_call(
    _tc_body,
    grid=(_NBLK,),
    in_specs=[
        pl.BlockSpec((_B, _EMB), lambda i: (0, 0)),
        pl.BlockSpec((_BLK, _EMB), lambda i: (i, 0)),
        pl.BlockSpec((_B, _EMB), lambda i: (0, 0)),
    ],
    out_specs=pl.BlockSpec((1, 1), lambda i: (0, 0)),
    out_shape=jax.ShapeDtypeStruct((1, 1), jnp.float32),
    scratch_shapes=[
        pltpu.VMEM((_B, _EMB), jnp.float32),
        pltpu.VMEM((_B, 1), jnp.float32),
        pltpu.VMEM((_B, 1), jnp.float32),
    ],
)


def kernel(local_embeddings, local_labels, weight):
    labels = local_labels.astype(jnp.int32)
    cand = jnp.asarray(_build_candidates())

    arange_b = jnp.arange(_B, dtype=jnp.int32)
    occ = jnp.zeros((_NUM_CLASSES,), jnp.int32).at[labels].set(1)
    s = jnp.sum(occ)
    padded = s < _B
    zero_was = occ[0]
    forced0 = padded & (zero_was == 0)
    u = s + forced0.astype(jnp.int32)
    is_pos = (occ[cand] == 1) | (forced0 & (cand == 0))
    nneg = _NUM_SAMPLE - u
    rank = jnp.cumsum((~is_pos).astype(jnp.int32)) - 1
    negpos = jnp.where((~is_pos) & (rank < nneg), rank, _NPAD)
    buf = jnp.zeros((_NPAD + 1,), jnp.int32).at[negpos].set(cand, mode="drop")
    slot = jnp.full((_NUM_CLASSES,), 2**30, jnp.int32).at[labels].min(arange_b)
    firstocc = slot[labels] == arange_b
    posrank = jnp.cumsum(firstocc.astype(jnp.int32)) - 1
    pospos = jnp.where(
        firstocc, nneg + forced0.astype(jnp.int32) + posrank, _NPAD
    )
    buf = buf.at[pospos].set(labels, mode="drop")

    gidx = jnp.concatenate([buf[:_NPAD], labels]).reshape(_NW, _RPW)
    rows = _sc_gather()(weight, gidx)

    loss = _tc_loss(local_embeddings, rows[:_NPAD], rows[_NPAD:])
    return loss.reshape(())
